# baseline (speedup 1.0000x reference)
import jax
import jax.numpy as jnp
from jax.experimental import pallas as pl
from jax.experimental.pallas import tpu as pltpu


def _loss_kernel(x_ref, m_ref, pos_ref, a_ref, out_ref, accg_ref, accG_ref):
    g = pl.program_id(0)
    C = x_ref.shape[1]
    Tn = x_ref.shape[0]
    Q = accG_ref.shape[1]
    R = Q // Tn

    @pl.when(g == 0)
    def _init():
        accg_ref[...] = jnp.zeros_like(accg_ref)
        accG_ref[...] = jnp.zeros_like(accG_ref)

    x = x_ref[...]
    cols = jax.lax.broadcasted_iota(jnp.int32, (1, C), 1)
    tgt = pos_ref[...] - g * C
    oh = (cols == tgt).astype(jnp.float32)
    accG_ref[...] += jax.lax.dot_general(
        x, oh, (((1,), (1,)), ((), ())), preferred_element_type=jnp.float32)

    bm = (m_ref[...] > 0).astype(jnp.float32)
    xb = x * bm
    x2b = x * xb
    lhs = jnp.concatenate([bm, xb, x2b], axis=0)
    rhs = jnp.concatenate([bm, xb], axis=0)
    accg_ref[...] += jax.lax.dot_general(
        lhs, rhs, (((1,), (1,)), ((), ())), preferred_element_type=jnp.float32)

    @pl.when(g == pl.num_programs(0) - 1)
    def _finish():
        S = accg_ref[...]
        n = S[0:Tn, 0:Tn]
        p_xb = S[Tn:2 * Tn, 0:Tn]
        p_xx = S[Tn:2 * Tn, Tn:2 * Tn]
        p_x2b = S[2 * Tn:3 * Tn, 0:Tn]

        s1 = p_xb - p_xb.T
        s2 = p_x2b - 2.0 * p_xx + p_x2b.T
        n1 = jnp.maximum(n, 1.0)
        var = (s2 - s1 * s1 / n1) / jnp.maximum(n - 1.0, 1.0)
        std = jnp.sqrt(jnp.maximum(var, 0.0))

        ii = jax.lax.broadcasted_iota(jnp.int32, (Tn, Tn), 0)
        jj = jax.lax.broadcasted_iota(jnp.int32, (Tn, Tn), 1)
        eye = ii == jj
        dcol = jnp.sum(jnp.where(eye, n, 0.0), axis=1, keepdims=True)
        drow = jnp.sum(jnp.where(eye, n, 0.0), axis=0, keepdims=True)
        diffcount = dcol + drow - 2.0 * n
        inf = jnp.float32(jnp.inf)
        scores = jnp.where((diffcount > 0.0) & (~eye), std, inf)

        Gall = accG_ref[...]
        qi = jax.lax.broadcasted_iota(jnp.int32, (Tn, Q), 0)
        qj = jax.lax.broadcasted_iota(jnp.int32, (Tn, Q), 1)
        qmask = (qj // R) == qi
        a_row = a_ref[...]

        cur = scores
        negs = []
        sels = []
        for _ in range(3):
            mval = jnp.min(cur, axis=1, keepdims=True)
            is_min = cur == mval
            idx = jnp.min(jnp.where(is_min, jj, Tn), axis=1, keepdims=True)
            sels.append((jj == idx).astype(jnp.float32))
            negs.append(-mval)
            cur = jnp.where(jj == idx, inf, cur)

        selcat = jnp.concatenate(sels, axis=0)
        vall = jax.lax.dot_general(
            selcat, Gall, (((1,), (0,)), ((), ())),
            preferred_element_type=jnp.float32)
        qmask3 = jnp.concatenate([qmask, qmask, qmask], axis=0)
        term = jnp.where(qmask3, jnp.abs(a_row - vall), 0.0)
        rows3 = jnp.sum(term, axis=1, keepdims=True)
        rs = jnp.concatenate(
            [rows3[0:Tn], rows3[Tn:2 * Tn], rows3[2 * Tn:3 * Tn]], axis=1)

        negcat = jnp.concatenate(negs, axis=1)
        mx = jnp.max(negcat, axis=1, keepdims=True)
        e = jnp.exp(negcat - mx)
        w = e / jnp.sum(e, axis=1, keepdims=True)
        per_row = jnp.sum(w * rs, axis=1, keepdims=True)
        out_ref[...] = jnp.sum(per_row, axis=0, keepdims=True)


def kernel(X, A, M, T, nM, row_elements_pos, max_time):
    Tn, D = X.shape
    R = row_elements_pos.shape[1]
    Q = Tn * R
    CHUNK = 8192
    grid = D // CHUNK

    pos = row_elements_pos.astype(jnp.int32).reshape(Q, 1)
    a_row = A.astype(jnp.float32).reshape(1, Q)

    out = pl.pallas_call(
        _loss_kernel,
        grid=(grid,),
        in_specs=[
            pl.BlockSpec((Tn, CHUNK), lambda g: (0, g)),
            pl.BlockSpec((Tn, CHUNK), lambda g: (0, g)),
            pl.BlockSpec((Q, 1), lambda g: (0, 0)),
            pl.BlockSpec((1, Q), lambda g: (0, 0)),
        ],
        out_specs=pl.BlockSpec((1, 1), lambda g: (0, 0)),
        out_shape=jax.ShapeDtypeStruct((1, 1), jnp.float32),
        scratch_shapes=[
            pltpu.VMEM((3 * Tn, 2 * Tn), jnp.float32),
            pltpu.VMEM((Tn, Q), jnp.float32),
        ],
    )(X, M, pos, a_row)
    return jnp.reshape(out, ())


# speedup vs baseline: 1.1949x; 1.1949x over previous
import jax
import jax.numpy as jnp
from jax.experimental import pallas as pl
from jax.experimental.pallas import tpu as pltpu


def _loss_kernel(x_ref, m_ref, pos_ref, a_ref, out_ref, accg_ref, accG_ref):
    g = pl.program_id(0)
    C = x_ref.shape[1]
    Tn = x_ref.shape[0]
    Q = accG_ref.shape[1]
    R = Q // Tn

    @pl.when(g == 0)
    def _init():
        accg_ref[...] = jnp.zeros_like(accg_ref)
        accG_ref[...] = jnp.zeros_like(accG_ref)

    x = x_ref[...]
    bm = (m_ref[...] > 0).astype(jnp.float32)
    xb = x * bm
    x2b = x * xb
    lhs = jnp.concatenate([bm, xb, x2b], axis=0)
    rhs = jnp.concatenate([bm, xb], axis=0)
    accg_ref[...] += jax.lax.dot_general(
        lhs, rhs, (((1,), (1,)), ((), ())), preferred_element_type=jnp.float32)

    rows = jax.lax.broadcasted_iota(jnp.int32, (C, 1), 0)
    tgt = pos_ref[...] - g * C
    oh = (rows == tgt).astype(jnp.float32)
    accG_ref[...] += jax.lax.dot_general(
        x, oh, (((1,), (0,)), ((), ())),
        preferred_element_type=jnp.float32)

    @pl.when(g == pl.num_programs(0) - 1)
    def _finish():
        S = accg_ref[...]
        n = S[0:Tn, 0:Tn]
        p_xb = S[Tn:2 * Tn, 0:Tn]
        p_xx = S[Tn:2 * Tn, Tn:2 * Tn]
        p_x2b = S[2 * Tn:3 * Tn, 0:Tn]

        s1 = p_xb - p_xb.T
        s2 = p_x2b - 2.0 * p_xx + p_x2b.T
        n1 = jnp.maximum(n, 1.0)
        var = (s2 - s1 * s1 / n1) / jnp.maximum(n - 1.0, 1.0)
        std = jnp.sqrt(jnp.maximum(var, 0.0))

        ii = jax.lax.broadcasted_iota(jnp.int32, (Tn, Tn), 0)
        jj = jax.lax.broadcasted_iota(jnp.int32, (Tn, Tn), 1)
        eye = ii == jj
        dcol = jnp.sum(jnp.where(eye, n, 0.0), axis=1, keepdims=True)
        drow = jnp.sum(jnp.where(eye, n, 0.0), axis=0, keepdims=True)
        diffcount = dcol + drow - 2.0 * n
        inf = jnp.float32(jnp.inf)
        scores = jnp.where((diffcount > 0.0) & (~eye), std, inf)

        Gall = accG_ref[...]
        qi = jax.lax.broadcasted_iota(jnp.int32, (Tn, Q), 0)
        qj = jax.lax.broadcasted_iota(jnp.int32, (Tn, Q), 1)
        qmask = (qj // R) == qi
        a_row = a_ref[...]

        cur = scores
        negs = []
        sels = []
        for _ in range(3):
            mval = jnp.min(cur, axis=1, keepdims=True)
            is_min = cur == mval
            idx = jnp.min(jnp.where(is_min, jj, Tn), axis=1, keepdims=True)
            sels.append((jj == idx).astype(jnp.float32))
            negs.append(-mval)
            cur = jnp.where(jj == idx, inf, cur)

        selcat = jnp.concatenate(sels, axis=0)
        vall = jax.lax.dot_general(
            selcat, Gall, (((1,), (0,)), ((), ())),
            preferred_element_type=jnp.float32)
        qmask3 = jnp.concatenate([qmask, qmask, qmask], axis=0)
        term = jnp.where(qmask3, jnp.abs(a_row - vall), 0.0)
        rows3 = jnp.sum(term, axis=1, keepdims=True)
        rs = jnp.concatenate(
            [rows3[0:Tn], rows3[Tn:2 * Tn], rows3[2 * Tn:3 * Tn]], axis=1)

        negcat = jnp.concatenate(negs, axis=1)
        mx = jnp.max(negcat, axis=1, keepdims=True)
        e = jnp.exp(negcat - mx)
        w = e / jnp.sum(e, axis=1, keepdims=True)
        per_row = jnp.sum(w * rs, axis=1, keepdims=True)
        out_ref[...] = jnp.sum(per_row, axis=0, keepdims=True)


def kernel(X, A, M, T, nM, row_elements_pos, max_time):
    Tn, D = X.shape
    R = row_elements_pos.shape[1]
    Q = Tn * R
    CHUNK = 8192
    grid = D // CHUNK

    pos = row_elements_pos.astype(jnp.int32).reshape(1, Q)
    a_row = A.astype(jnp.float32).reshape(1, Q)

    out = pl.pallas_call(
        _loss_kernel,
        grid=(grid,),
        in_specs=[
            pl.BlockSpec((Tn, CHUNK), lambda g: (0, g)),
            pl.BlockSpec((Tn, CHUNK), lambda g: (0, g)),
            pl.BlockSpec((1, Q), lambda g: (0, 0)),
            pl.BlockSpec((1, Q), lambda g: (0, 0)),
        ],
        out_specs=pl.BlockSpec((1, 1), lambda g: (0, 0)),
        out_shape=jax.ShapeDtypeStruct((1, 1), jnp.float32),
        scratch_shapes=[
            pltpu.VMEM((3 * Tn, 2 * Tn), jnp.float32),
            pltpu.VMEM((Tn, Q), jnp.float32),
        ],
    )(X, M, pos, a_row)
    return jnp.reshape(out, ())


# speedup vs baseline: 1.2120x; 1.0144x over previous
import jax
import jax.numpy as jnp
from jax.experimental import pallas as pl
from jax.experimental.pallas import tpu as pltpu


def _loss_kernel(x_ref, m_ref, pos_ref, a_ref, out_ref, accg_ref, accG_ref):
    g = pl.program_id(0)
    C = x_ref.shape[1]
    Tn = x_ref.shape[0]
    Q = accG_ref.shape[1]
    R = Q // Tn

    @pl.when(g == 0)
    def _init():
        accg_ref[...] = jnp.zeros_like(accg_ref)
        accG_ref[...] = jnp.zeros_like(accG_ref)

    x = x_ref[...]
    bm = (m_ref[...] > 0).astype(jnp.float32)
    xb = x * bm
    x2b = x * xb
    lhs = jnp.concatenate([bm, xb, x2b], axis=0)
    rhs = jnp.concatenate([bm, xb], axis=0)
    accg_ref[...] += jax.lax.dot_general(
        lhs, rhs, (((1,), (1,)), ((), ())), preferred_element_type=jnp.float32)

    rows = jax.lax.broadcasted_iota(jnp.int32, (C, 1), 0)
    tgt = pos_ref[...] - g * C
    oh = (rows == tgt).astype(jnp.float32)
    accG_ref[...] += jax.lax.dot_general(
        x, oh, (((1,), (0,)), ((), ())),
        preferred_element_type=jnp.float32)

    @pl.when(g == pl.num_programs(0) - 1)
    def _finish():
        S = accg_ref[...]
        n = S[0:Tn, 0:Tn]
        p_xb = S[Tn:2 * Tn, 0:Tn]
        p_xx = S[Tn:2 * Tn, Tn:2 * Tn]
        p_x2b = S[2 * Tn:3 * Tn, 0:Tn]

        s1 = p_xb - p_xb.T
        s2 = p_x2b - 2.0 * p_xx + p_x2b.T
        n1 = jnp.maximum(n, 1.0)
        var = (s2 - s1 * s1 / n1) / jnp.maximum(n - 1.0, 1.0)
        std = jnp.sqrt(jnp.maximum(var, 0.0))

        ii = jax.lax.broadcasted_iota(jnp.int32, (Tn, Tn), 0)
        jj = jax.lax.broadcasted_iota(jnp.int32, (Tn, Tn), 1)
        eye = ii == jj
        dcol = jnp.sum(jnp.where(eye, n, 0.0), axis=1, keepdims=True)
        drow = jnp.sum(jnp.where(eye, n, 0.0), axis=0, keepdims=True)
        diffcount = dcol + drow - 2.0 * n
        inf = jnp.float32(jnp.inf)
        scores = jnp.where((diffcount > 0.0) & (~eye), std, inf)

        Gall = accG_ref[...]
        qi = jax.lax.broadcasted_iota(jnp.int32, (Tn, Q), 0)
        qj = jax.lax.broadcasted_iota(jnp.int32, (Tn, Q), 1)
        qmaskf = ((qj // R) == qi).astype(jnp.float32)
        a_row = a_ref[...]
        B = jnp.abs(a_row - Gall)
        term = jax.lax.dot_general(
            qmaskf, B, (((1,), (1,)), ((), ())),
            preferred_element_type=jnp.float32)

        cur = scores
        negs = []
        rowsums = []
        for _ in range(3):
            mval = jnp.min(cur, axis=1, keepdims=True)
            is_min = cur == mval
            idx = jnp.min(jnp.where(is_min, jj, Tn), axis=1, keepdims=True)
            sel = (jj == idx).astype(jnp.float32)
            rowsums.append(jnp.sum(sel * term, axis=1, keepdims=True))
            negs.append(-mval)
            cur = jnp.where(jj == idx, inf, cur)

        rs = jnp.concatenate(rowsums, axis=1)
        negcat = jnp.concatenate(negs, axis=1)
        mx = jnp.max(negcat, axis=1, keepdims=True)
        e = jnp.exp(negcat - mx)
        w = e / jnp.sum(e, axis=1, keepdims=True)
        per_row = jnp.sum(w * rs, axis=1, keepdims=True)
        out_ref[...] = jnp.sum(per_row, axis=0, keepdims=True)


def kernel(X, A, M, T, nM, row_elements_pos, max_time):
    Tn, D = X.shape
    R = row_elements_pos.shape[1]
    Q = Tn * R
    CHUNK = 8192
    grid = D // CHUNK

    pos = row_elements_pos.astype(jnp.int32).reshape(1, Q)
    a_row = A.astype(jnp.float32).reshape(1, Q)

    out = pl.pallas_call(
        _loss_kernel,
        grid=(grid,),
        in_specs=[
            pl.BlockSpec((Tn, CHUNK), lambda g: (0, g)),
            pl.BlockSpec((Tn, CHUNK), lambda g: (0, g)),
            pl.BlockSpec((1, Q), lambda g: (0, 0)),
            pl.BlockSpec((1, Q), lambda g: (0, 0)),
        ],
        out_specs=pl.BlockSpec((1, 1), lambda g: (0, 0)),
        out_shape=jax.ShapeDtypeStruct((1, 1), jnp.float32),
        scratch_shapes=[
            pltpu.VMEM((3 * Tn, 2 * Tn), jnp.float32),
            pltpu.VMEM((Tn, Q), jnp.float32),
        ],
    )(X, M, pos, a_row)
    return jnp.reshape(out, ())


# speedup vs baseline: 1.2160x; 1.0033x over previous
import jax
import jax.numpy as jnp
from jax.experimental import pallas as pl
from jax.experimental.pallas import tpu as pltpu


def _loss_kernel(x_ref, m_ref, pos_ref, a_ref, out_ref, accg_ref, accG_ref):
    g = pl.program_id(0)
    C = x_ref.shape[1]
    Tn = x_ref.shape[0]
    Q = accG_ref.shape[1]
    R = Q // Tn

    @pl.when(g == 0)
    def _init():
        accg_ref[...] = jnp.zeros_like(accg_ref)
        accG_ref[...] = jnp.zeros_like(accG_ref)

    x = x_ref[...]
    bm = (m_ref[...] > 0).astype(jnp.float32)
    xb = x * bm
    x2b = x * xb
    lhs = jnp.concatenate([bm, xb, x2b], axis=0)
    rhs = jnp.concatenate([bm, xb], axis=0)
    accg_ref[...] += jax.lax.dot_general(
        lhs, rhs, (((1,), (1,)), ((), ())), preferred_element_type=jnp.float32)

    rows = jax.lax.broadcasted_iota(jnp.int32, (C, 1), 0)
    tgt = pos_ref[...] - g * C
    oh = (rows == tgt).astype(jnp.float32)
    accG_ref[...] += jax.lax.dot_general(
        x, oh, (((1,), (0,)), ((), ())),
        preferred_element_type=jnp.float32)

    @pl.when(g == pl.num_programs(0) - 1)
    def _finish():
        S = accg_ref[...]
        n = S[0:Tn, 0:Tn]
        p_xb = S[Tn:2 * Tn, 0:Tn]
        p_xx = S[Tn:2 * Tn, Tn:2 * Tn]
        p_x2b = S[2 * Tn:3 * Tn, 0:Tn]

        s1 = p_xb - p_xb.T
        s2 = p_x2b - 2.0 * p_xx + p_x2b.T
        n1 = jnp.maximum(n, 1.0)
        var = (s2 - s1 * s1 / n1) / jnp.maximum(n - 1.0, 1.0)
        std = jnp.sqrt(jnp.maximum(var, 0.0))

        ii = jax.lax.broadcasted_iota(jnp.int32, (Tn, Tn), 0)
        jj = jax.lax.broadcasted_iota(jnp.int32, (Tn, Tn), 1)
        eye = ii == jj
        dcol = jnp.sum(jnp.where(eye, n, 0.0), axis=1, keepdims=True)
        drow = jnp.sum(jnp.where(eye, n, 0.0), axis=0, keepdims=True)
        diffcount = dcol + drow - 2.0 * n
        inf = jnp.float32(jnp.inf)
        scores = jnp.where((diffcount > 0.0) & (~eye), std, inf)

        Gall = accG_ref[...]
        qi = jax.lax.broadcasted_iota(jnp.int32, (Tn, Q), 0)
        qj = jax.lax.broadcasted_iota(jnp.int32, (Tn, Q), 1)
        qmaskf = ((qj // R) == qi).astype(jnp.float32)
        a_row = a_ref[...]
        B = jnp.abs(a_row - Gall)
        term = jax.lax.dot_general(
            qmaskf, B, (((1,), (1,)), ((), ())),
            preferred_element_type=jnp.float32)

        cur = scores
        negs = []
        rowsums = []
        for _ in range(3):
            mval = jnp.min(cur, axis=1, keepdims=True)
            is_min = cur == mval
            idx = jnp.min(jnp.where(is_min, jj, Tn), axis=1, keepdims=True)
            sel = (jj == idx).astype(jnp.float32)
            rowsums.append(jnp.sum(sel * term, axis=1, keepdims=True))
            negs.append(-mval)
            cur = jnp.where(jj == idx, inf, cur)

        rs = jnp.concatenate(rowsums, axis=1)
        negcat = jnp.concatenate(negs, axis=1)
        mx = jnp.max(negcat, axis=1, keepdims=True)
        e = jnp.exp(negcat - mx)
        w = e / jnp.sum(e, axis=1, keepdims=True)
        per_row = jnp.sum(w * rs, axis=1, keepdims=True)
        out_ref[...] = jnp.sum(per_row, axis=0, keepdims=True)


def kernel(X, A, M, T, nM, row_elements_pos, max_time):
    Tn, D = X.shape
    R = row_elements_pos.shape[1]
    Q = Tn * R
    CHUNK = 16384
    grid = D // CHUNK

    pos = row_elements_pos.astype(jnp.int32).reshape(1, Q)
    a_row = A.astype(jnp.float32).reshape(1, Q)

    out = pl.pallas_call(
        _loss_kernel,
        grid=(grid,),
        in_specs=[
            pl.BlockSpec((Tn, CHUNK), lambda g: (0, g)),
            pl.BlockSpec((Tn, CHUNK), lambda g: (0, g)),
            pl.BlockSpec((1, Q), lambda g: (0, 0)),
            pl.BlockSpec((1, Q), lambda g: (0, 0)),
        ],
        out_specs=pl.BlockSpec((1, 1), lambda g: (0, 0)),
        out_shape=jax.ShapeDtypeStruct((1, 1), jnp.float32),
        scratch_shapes=[
            pltpu.VMEM((3 * Tn, 2 * Tn), jnp.float32),
            pltpu.VMEM((Tn, Q), jnp.float32),
        ],
    )(X, M, pos, a_row)
    return jnp.reshape(out, ())
